# baseline (speedup 1.0000x reference)
import functools

import jax
import jax.numpy as jnp
from jax import lax
from jax.experimental import pallas as pl
from jax.experimental.pallas import tpu as pltpu
from jax.experimental.pallas import tpu_sc as plsc

N = 10000
E = 160000
D = 256
H = 16
C = 2
R = 4

NC, NS, L = 2, 16, 16
NW = NC * NS
NPAD = 10240
NR = R * NPAD
GARB = NR - 8
EPT = E // NW
CH = 128
NCH = (EPT + CH - 1) // CH
EBUF = NCH * CH
BPT = NR // NS
NBLK = 1024
PBLK = NBLK // 8
NG = NPAD // NBLK
PROW = NPAD // 8
CP = 128
NBUF = 4


def _pack(h):
    hs = h.reshape(PBLK, 8, H)
    return jnp.concatenate([hs[:, j, :] for j in range(8)], axis=1)


_mesh = plsc.VectorSubcoreMesh(core_axis_name="c", subcore_axis_name="s")


def _tca_body(x_ref, w_ref, b1_ref, h1_ref, xr_ref):
    h = jnp.dot(x_ref[...], w_ref[...], preferred_element_type=jnp.float32)
    for r in range(R):
        h1_ref[r] = _pack(h[:, H * r:H * (r + 1)])
    xr_ref[...] = _pack(h[:, H * R:H * (R + 1)] + b1_ref[...])


def _tca(x, wcat, b1):
    return pl.pallas_call(
        _tca_body,
        grid=(NG,),
        in_specs=[
            pl.BlockSpec((NBLK, D), lambda i: (i, 0)),
            pl.BlockSpec((D, H * (R + 1)), lambda i: (0, 0)),
            pl.BlockSpec((1, H), lambda i: (0, 0)),
        ],
        out_specs=[
            pl.BlockSpec((R, PBLK, 128), lambda i: (0, i, 0)),
            pl.BlockSpec((PBLK, 128), lambda i: (i, 0)),
        ],
        out_shape=[
            jax.ShapeDtypeStruct((R, PROW, 128), jnp.float32),
            jax.ShapeDtypeStruct((PROW, 128), jnp.float32),
        ],
    )(x, wcat, b1)


@functools.partial(
    pl.kernel,
    out_type=[
        jax.ShapeDtypeStruct((NC, NR, H), jnp.float32),
        jax.ShapeDtypeStruct((NC, NR, H), jnp.float32),
    ],
    mesh=_mesh,
    scratch_types=[
        pltpu.VMEM_SHARED((NR, H), jnp.float32),
        pltpu.VMEM_SHARED((NR, H), jnp.float32),
        pltpu.VMEM((EBUF,), jnp.int32),
        pltpu.VMEM((EBUF,), jnp.int32),
        pltpu.VMEM((EBUF,), jnp.int32),
        pltpu.VMEM((NCH, CH), jnp.int32),
        pltpu.VMEM((NCH, CH), jnp.int32),
        [pltpu.VMEM((CH, H), jnp.float32) for _ in range(NBUF)],
        pltpu.VMEM((CH, H), jnp.float32),
        [pltpu.SemaphoreType.DMA for _ in range(NBUF)],
    ],
    compiler_params=pltpu.CompilerParams(use_tc_tiling_on_sc=False),
)
def _sc1(h1_hbm, ei_hbm, et_hbm, zeros_hbm, ones_hbm,
         a_out, c_out,
         a_sh, c_sh, src_v, dst_v, et_v, ga2, key2, rows, ones_v, gsem):
    cid = lax.axis_index("c")
    sid = lax.axis_index("s")
    base = (cid * NS + sid) * EPT
    rows0 = sid * BPT
    dz1 = pltpu.async_copy(zeros_hbm, a_sh.at[pl.ds(rows0, BPT)], gsem[0])
    dz2 = pltpu.async_copy(zeros_hbm, c_sh.at[pl.ds(rows0, BPT)], gsem[1])
    ds_ = pltpu.async_copy(ei_hbm.at[0, pl.ds(base, EPT)],
                           src_v.at[pl.ds(0, EPT)], gsem[2])
    dd_ = pltpu.async_copy(ei_hbm.at[1, pl.ds(base, EPT)],
                           dst_v.at[pl.ds(0, EPT)], gsem[3])
    pltpu.sync_copy(et_hbm.at[pl.ds(base, EPT)], et_v.at[pl.ds(0, EPT)])
    pltpu.sync_copy(ones_hbm, ones_v)
    ds_.wait()
    dd_.wait()

    lane = lax.iota(jnp.int32, L)

    def idx_chunk(c, carry):
        off = c * CH
        for i in range(CH // L):
            pos = off + i * L
            valid = (lane + pos) < EPT
            s16 = src_v[pl.ds(pos, L)]
            d16 = dst_v[pl.ds(pos, L)]
            tb = et_v[pl.ds(pos, L)] * NPAD
            ga2[c, pl.ds(i * L, L)] = jnp.where(valid, tb + s16, 0)
            key2[c, pl.ds(i * L, L)] = jnp.where(valid, tb + d16, GARB)
        return carry

    lax.fori_loop(0, NCH, idx_chunk, 0)
    dz1.wait()
    dz2.wait()
    plsc.subcore_barrier()

    for b in range(NBUF):
        pltpu.async_copy(h1_hbm.at[ga2.at[b]], rows[b], gsem[b])

    def stream_chunk(j, carry):
        for b in range(NBUF):
            c = j * NBUF + b
            pltpu.make_async_copy(
                h1_hbm.at[ga2.at[c]], rows[b], gsem[b]).wait()
            pltpu.sync_copy(ones_v, c_sh.at[key2.at[c]], add=True)
            pltpu.sync_copy(rows[b], a_sh.at[key2.at[c]], add=True)

            @pl.when(j < NCH // NBUF - 1)
            def _():
                pltpu.async_copy(h1_hbm.at[ga2.at[c + NBUF]], rows[b], gsem[b])
        return carry

    lax.fori_loop(0, NCH // NBUF, stream_chunk, 0)
    plsc.subcore_barrier()
    pltpu.sync_copy(a_sh.at[pl.ds(rows0, BPT)],
                    a_out.at[cid, pl.ds(rows0, BPT)])
    pltpu.sync_copy(c_sh.at[pl.ds(rows0, BPT)],
                    c_out.at[cid, pl.ds(rows0, BPT)])


def _tcb_body(a_ref, c_ref, xr_ref, out1_ref, inv_ref):
    cnt = c_ref[0] + c_ref[1]
    inv = 1.0 / jnp.maximum(cnt, 1.0)
    agg = jnp.sum(inv * (a_ref[0] + a_ref[1]), axis=0)
    out1_ref[...] = jnp.maximum(agg + xr_ref[...], 0.0)
    inv_ref[...] = inv


def _tcb(a1, ccnt, xr1):
    return pl.pallas_call(
        _tcb_body,
        grid=(NG,),
        in_specs=[
            pl.BlockSpec((NC, R, PBLK, 128), lambda i: (0, 0, i, 0)),
            pl.BlockSpec((NC, R, PBLK, 128), lambda i: (0, 0, i, 0)),
            pl.BlockSpec((PBLK, 128), lambda i: (i, 0)),
        ],
        out_specs=[
            pl.BlockSpec((PBLK, 128), lambda i: (i, 0)),
            pl.BlockSpec((R, PBLK, 128), lambda i: (0, i, 0)),
        ],
        out_shape=[
            jax.ShapeDtypeStruct((PROW, 128), jnp.float32),
            jax.ShapeDtypeStruct((R, PROW, 128), jnp.float32),
        ],
    )(a1, ccnt, xr1)


@functools.partial(
    pl.kernel,
    out_type=jax.ShapeDtypeStruct((NC, NR, H), jnp.float32),
    mesh=_mesh,
    scratch_types=[
        pltpu.VMEM_SHARED((NR, H), jnp.float32),
        pltpu.VMEM((EBUF,), jnp.int32),
        pltpu.VMEM((EBUF,), jnp.int32),
        pltpu.VMEM((EBUF,), jnp.int32),
        pltpu.VMEM((NCH, CH), jnp.int32),
        pltpu.VMEM((NCH, CH), jnp.int32),
        [pltpu.VMEM((CH, H), jnp.float32) for _ in range(NBUF)],
        [pltpu.SemaphoreType.DMA for _ in range(NBUF)],
    ],
    compiler_params=pltpu.CompilerParams(use_tc_tiling_on_sc=False),
)
def _sc2(out1_hbm, ei_hbm, et_hbm, zeros_hbm,
         a_out,
         a_sh, src_v, dst_v, et_v, ga2, key2, rows, gsem):
    cid = lax.axis_index("c")
    sid = lax.axis_index("s")
    base = (cid * NS + sid) * EPT
    rows0 = sid * BPT
    dz1 = pltpu.async_copy(zeros_hbm, a_sh.at[pl.ds(rows0, BPT)], gsem[0])
    ds_ = pltpu.async_copy(ei_hbm.at[0, pl.ds(base, EPT)],
                           src_v.at[pl.ds(0, EPT)], gsem[1])
    dd_ = pltpu.async_copy(ei_hbm.at[1, pl.ds(base, EPT)],
                           dst_v.at[pl.ds(0, EPT)], gsem[2])
    pltpu.sync_copy(et_hbm.at[pl.ds(base, EPT)], et_v.at[pl.ds(0, EPT)])
    ds_.wait()
    dd_.wait()

    lane = lax.iota(jnp.int32, L)

    def idx_chunk(c, carry):
        off = c * CH
        for i in range(CH // L):
            pos = off + i * L
            valid = (lane + pos) < EPT
            s16 = src_v[pl.ds(pos, L)]
            d16 = dst_v[pl.ds(pos, L)]
            tb = et_v[pl.ds(pos, L)] * NPAD
            ga2[c, pl.ds(i * L, L)] = jnp.where(valid, s16, 0)
            key2[c, pl.ds(i * L, L)] = jnp.where(valid, tb + d16, GARB)
        return carry

    lax.fori_loop(0, NCH, idx_chunk, 0)
    dz1.wait()
    plsc.subcore_barrier()

    for b in range(NBUF):
        pltpu.async_copy(out1_hbm.at[ga2.at[b]], rows[b], gsem[b])

    def stream_chunk(j, carry):
        for b in range(NBUF):
            c = j * NBUF + b
            pltpu.make_async_copy(
                out1_hbm.at[ga2.at[c]], rows[b], gsem[b]).wait()
            pltpu.sync_copy(rows[b], a_sh.at[key2.at[c]], add=True)

            @pl.when(j < NCH // NBUF - 1)
            def _():
                pltpu.async_copy(
                    out1_hbm.at[ga2.at[c + NBUF]], rows[b], gsem[b])
        return carry

    lax.fori_loop(0, NCH // NBUF, stream_chunk, 0)
    plsc.subcore_barrier()
    pltpu.sync_copy(a_sh.at[pl.ds(rows0, BPT)],
                    a_out.at[cid, pl.ds(rows0, BPT)])


def _tcc_body(a_ref, inv_ref, out1_ref, w_ref, b2_ref, o_ref):
    p = inv_ref[...] * (a_ref[0] + a_ref[1])
    xcat = jnp.concatenate(
        [p[r] for r in range(R)] + [out1_ref[...]], axis=1)
    acc = jnp.dot(xcat, w_ref[...],
                  preferred_element_type=jnp.float32) + b2_ref[...]
    lanemod = lax.broadcasted_iota(jnp.int32, (PBLK, 128), 1) % H
    nxt = pltpu.roll(acc, 127, 1)
    m = jnp.maximum(acc, nxt)
    s = jnp.exp(acc - m) + jnp.exp(nxt - m)
    lse0 = m + jnp.log(s)
    lse = jnp.where(lanemod == 0, lse0, pltpu.roll(lse0, 1, 1))
    o_ref[...] = jnp.where(lanemod < C, acc - lse, 0.0)


def _tcc(a2, inv, out1, wstack, b2row):
    return pl.pallas_call(
        _tcc_body,
        grid=(NG,),
        in_specs=[
            pl.BlockSpec((NC, R, PBLK, 128), lambda i: (0, 0, i, 0)),
            pl.BlockSpec((R, PBLK, 128), lambda i: (0, i, 0)),
            pl.BlockSpec((PBLK, 128), lambda i: (i, 0)),
            pl.BlockSpec((128 * (R + 1), 128), lambda i: (0, 0)),
            pl.BlockSpec((1, 128), lambda i: (0, 0)),
        ],
        out_specs=pl.BlockSpec((PBLK, 128), lambda i: (i, 0)),
        out_shape=jax.ShapeDtypeStruct((PROW, 128), jnp.float32),
    )(a2, inv, out1, wstack, b2row)


def kernel(x, edge_index, edge_type, W1, root1, b1, W2, root2, b2):
    f32 = jnp.float32
    x = x.astype(f32)
    ei = edge_index.astype(jnp.int32)
    et = edge_type.astype(jnp.int32)

    zeros_t = jnp.zeros((BPT, H), f32)
    ones_t = jnp.ones((CH, H), f32)

    wcat = jnp.concatenate(
        [W1.astype(f32)[r] for r in range(R)] + [root1.astype(f32)], axis=1)

    h1p, xr1p = _tca(x, wcat, b1.astype(f32).reshape(1, H))

    a1, cc = _sc1(h1p.reshape(NR, H), ei, et, zeros_t, ones_t)

    out1p, invp = _tcb(a1.reshape(NC, R, PROW, 128),
                       cc.reshape(NC, R, PROW, 128), xr1p)

    a2 = _sc2(out1p.reshape(NPAD, H), ei, et, zeros_t)

    eye8 = jnp.eye(8, dtype=f32)
    blocks = [jnp.kron(eye8, jnp.pad(W2.astype(f32)[r], ((0, 0), (0, H - C))))
              for r in range(R)]
    blocks.append(jnp.kron(eye8, jnp.pad(root2.astype(f32),
                                         ((0, 0), (0, H - C)))))
    wstack = jnp.concatenate(blocks, axis=0)
    b2row = jnp.tile(jnp.pad(b2.astype(f32), (0, H - C)), 8).reshape(1, 128)

    out = _tcc(a2.reshape(NC, R, PROW, 128), invp, out1p, wstack, b2row)
    return out.reshape(NPAD, H)[:N, :C]


# speedup vs baseline: 1.0164x; 1.0164x over previous
import functools

import jax
import jax.numpy as jnp
from jax import lax
from jax.experimental import pallas as pl
from jax.experimental.pallas import tpu as pltpu
from jax.experimental.pallas import tpu_sc as plsc

N = 10000
E = 160000
D = 256
H = 16
C = 2
R = 4

NC, NS, L = 2, 16, 16
NW = NC * NS
NPAD = 10240
NR = R * NPAD
GARB = NR - 8
EPT = E // NW
CH = 128
NCH = (EPT + CH - 1) // CH
EBUF = NCH * CH
BPT = NR // NS
NBLK = 1024
PBLK = NBLK // 8
NG = NPAD // NBLK
PROW = NPAD // 8
CP = 128
NBUF = 4
NROW = 8


def _stream_pass(table_hbm, ga2, key2, rows, gsem, ssem, a_sh,
                 c_sh=None, ones_v=None, osem=None):
    for b in range(NBUF):
        pltpu.async_copy(table_hbm.at[ga2.at[b]], rows[b], gsem[b])

    def body(j, carry):
        for b in range(NROW):
            c = j * NROW + b
            pltpu.make_async_copy(
                table_hbm.at[ga2.at[c]], rows[b], gsem[b]).wait()
            if c_sh is not None:
                @pl.when(c >= NROW)
                def _():
                    pltpu.make_async_copy(
                        ones_v, c_sh.at[key2.at[c - NROW]], osem[b]).wait()
                pltpu.async_copy(ones_v, c_sh.at[key2.at[c]], osem[b],
                                 add=True)
            pltpu.async_copy(rows[b], a_sh.at[key2.at[c]], ssem[b], add=True)
            bp = (b + NBUF) % NROW

            @pl.when((c + NBUF < NCH) & (c >= NBUF))
            def _():
                pltpu.make_async_copy(
                    rows[bp], a_sh.at[key2.at[c - NBUF]], ssem[bp]).wait()

            @pl.when(c + NBUF < NCH)
            def _():
                pltpu.async_copy(
                    table_hbm.at[ga2.at[c + NBUF]], rows[bp], gsem[bp])
        return carry

    lax.fori_loop(0, NCH // NROW, body, 0)
    for b in range(NROW):
        c = NCH - NROW + b
        pltpu.make_async_copy(rows[b], a_sh.at[key2.at[c]], ssem[b]).wait()
        if c_sh is not None:
            pltpu.make_async_copy(
                ones_v, c_sh.at[key2.at[c]], osem[b]).wait()


def _pack(h):
    hs = h.reshape(PBLK, 8, H)
    return jnp.concatenate([hs[:, j, :] for j in range(8)], axis=1)


_mesh = plsc.VectorSubcoreMesh(core_axis_name="c", subcore_axis_name="s")


def _tca_body(x_ref, w_ref, b1_ref, h1_ref, xr_ref):
    h = jnp.dot(x_ref[...], w_ref[...], preferred_element_type=jnp.float32)
    for r in range(R):
        h1_ref[r] = _pack(h[:, H * r:H * (r + 1)])
    xr_ref[...] = _pack(h[:, H * R:H * (R + 1)] + b1_ref[...])


def _tca(x, wcat, b1):
    return pl.pallas_call(
        _tca_body,
        grid=(NG,),
        in_specs=[
            pl.BlockSpec((NBLK, D), lambda i: (i, 0)),
            pl.BlockSpec((D, H * (R + 1)), lambda i: (0, 0)),
            pl.BlockSpec((1, H), lambda i: (0, 0)),
        ],
        out_specs=[
            pl.BlockSpec((R, PBLK, 128), lambda i: (0, i, 0)),
            pl.BlockSpec((PBLK, 128), lambda i: (i, 0)),
        ],
        out_shape=[
            jax.ShapeDtypeStruct((R, PROW, 128), jnp.float32),
            jax.ShapeDtypeStruct((PROW, 128), jnp.float32),
        ],
    )(x, wcat, b1)


@functools.partial(
    pl.kernel,
    out_type=[
        jax.ShapeDtypeStruct((NC, NR, H), jnp.float32),
        jax.ShapeDtypeStruct((NC, NR, H), jnp.float32),
    ],
    mesh=_mesh,
    scratch_types=[
        pltpu.VMEM_SHARED((NR, H), jnp.float32),
        pltpu.VMEM_SHARED((NR, H), jnp.float32),
        pltpu.VMEM((EBUF,), jnp.int32),
        pltpu.VMEM((EBUF,), jnp.int32),
        pltpu.VMEM((EBUF,), jnp.int32),
        pltpu.VMEM((NCH, CH), jnp.int32),
        pltpu.VMEM((NCH, CH), jnp.int32),
        [pltpu.VMEM((CH, H), jnp.float32) for _ in range(NROW)],
        pltpu.VMEM((CH, H), jnp.float32),
        [pltpu.SemaphoreType.DMA for _ in range(NROW)],
        [pltpu.SemaphoreType.DMA for _ in range(NROW)],
        [pltpu.SemaphoreType.DMA for _ in range(NROW)],
    ],
    compiler_params=pltpu.CompilerParams(use_tc_tiling_on_sc=False),
)
def _sc1(h1_hbm, ei_hbm, et_hbm, zeros_hbm, ones_hbm,
         a_out, c_out,
         a_sh, c_sh, src_v, dst_v, et_v, ga2, key2, rows, ones_v,
         gsem, ssem, osem):
    cid = lax.axis_index("c")
    sid = lax.axis_index("s")
    base = (cid * NS + sid) * EPT
    rows0 = sid * BPT
    dz1 = pltpu.async_copy(zeros_hbm, a_sh.at[pl.ds(rows0, BPT)], gsem[0])
    dz2 = pltpu.async_copy(zeros_hbm, c_sh.at[pl.ds(rows0, BPT)], gsem[1])
    ds_ = pltpu.async_copy(ei_hbm.at[0, pl.ds(base, EPT)],
                           src_v.at[pl.ds(0, EPT)], gsem[2])
    dd_ = pltpu.async_copy(ei_hbm.at[1, pl.ds(base, EPT)],
                           dst_v.at[pl.ds(0, EPT)], gsem[3])
    pltpu.sync_copy(et_hbm.at[pl.ds(base, EPT)], et_v.at[pl.ds(0, EPT)])
    pltpu.sync_copy(ones_hbm, ones_v)
    ds_.wait()
    dd_.wait()

    lane = lax.iota(jnp.int32, L)

    def idx_chunk(c, carry):
        off = c * CH
        for i in range(CH // L):
            pos = off + i * L
            valid = (lane + pos) < EPT
            s16 = src_v[pl.ds(pos, L)]
            d16 = dst_v[pl.ds(pos, L)]
            tb = et_v[pl.ds(pos, L)] * NPAD
            ga2[c, pl.ds(i * L, L)] = jnp.where(valid, tb + s16, 0)
            key2[c, pl.ds(i * L, L)] = jnp.where(valid, tb + d16, GARB)
        return carry

    lax.fori_loop(0, NCH, idx_chunk, 0)
    dz1.wait()
    dz2.wait()
    plsc.subcore_barrier()
    _stream_pass(h1_hbm, ga2, key2, rows, gsem, ssem, a_sh,
                 c_sh=c_sh, ones_v=ones_v, osem=osem)
    plsc.subcore_barrier()
    pltpu.sync_copy(a_sh.at[pl.ds(rows0, BPT)],
                    a_out.at[cid, pl.ds(rows0, BPT)])
    pltpu.sync_copy(c_sh.at[pl.ds(rows0, BPT)],
                    c_out.at[cid, pl.ds(rows0, BPT)])


def _tcb_body(a_ref, c_ref, xr_ref, out1_ref, inv_ref):
    cnt = c_ref[0] + c_ref[1]
    inv = 1.0 / jnp.maximum(cnt, 1.0)
    agg = jnp.sum(inv * (a_ref[0] + a_ref[1]), axis=0)
    out1_ref[...] = jnp.maximum(agg + xr_ref[...], 0.0)
    inv_ref[...] = inv


def _tcb(a1, ccnt, xr1):
    return pl.pallas_call(
        _tcb_body,
        grid=(NG,),
        in_specs=[
            pl.BlockSpec((NC, R, PBLK, 128), lambda i: (0, 0, i, 0)),
            pl.BlockSpec((NC, R, PBLK, 128), lambda i: (0, 0, i, 0)),
            pl.BlockSpec((PBLK, 128), lambda i: (i, 0)),
        ],
        out_specs=[
            pl.BlockSpec((PBLK, 128), lambda i: (i, 0)),
            pl.BlockSpec((R, PBLK, 128), lambda i: (0, i, 0)),
        ],
        out_shape=[
            jax.ShapeDtypeStruct((PROW, 128), jnp.float32),
            jax.ShapeDtypeStruct((R, PROW, 128), jnp.float32),
        ],
    )(a1, ccnt, xr1)


@functools.partial(
    pl.kernel,
    out_type=jax.ShapeDtypeStruct((NC, NR, H), jnp.float32),
    mesh=_mesh,
    scratch_types=[
        pltpu.VMEM_SHARED((NR, H), jnp.float32),
        pltpu.VMEM((EBUF,), jnp.int32),
        pltpu.VMEM((EBUF,), jnp.int32),
        pltpu.VMEM((EBUF,), jnp.int32),
        pltpu.VMEM((NCH, CH), jnp.int32),
        pltpu.VMEM((NCH, CH), jnp.int32),
        [pltpu.VMEM((CH, H), jnp.float32) for _ in range(NROW)],
        [pltpu.SemaphoreType.DMA for _ in range(NROW)],
        [pltpu.SemaphoreType.DMA for _ in range(NROW)],
    ],
    compiler_params=pltpu.CompilerParams(use_tc_tiling_on_sc=False),
)
def _sc2(out1_hbm, ei_hbm, et_hbm, zeros_hbm,
         a_out,
         a_sh, src_v, dst_v, et_v, ga2, key2, rows, gsem, ssem):
    cid = lax.axis_index("c")
    sid = lax.axis_index("s")
    base = (cid * NS + sid) * EPT
    rows0 = sid * BPT
    dz1 = pltpu.async_copy(zeros_hbm, a_sh.at[pl.ds(rows0, BPT)], gsem[0])
    ds_ = pltpu.async_copy(ei_hbm.at[0, pl.ds(base, EPT)],
                           src_v.at[pl.ds(0, EPT)], gsem[1])
    dd_ = pltpu.async_copy(ei_hbm.at[1, pl.ds(base, EPT)],
                           dst_v.at[pl.ds(0, EPT)], gsem[2])
    pltpu.sync_copy(et_hbm.at[pl.ds(base, EPT)], et_v.at[pl.ds(0, EPT)])
    ds_.wait()
    dd_.wait()

    lane = lax.iota(jnp.int32, L)

    def idx_chunk(c, carry):
        off = c * CH
        for i in range(CH // L):
            pos = off + i * L
            valid = (lane + pos) < EPT
            s16 = src_v[pl.ds(pos, L)]
            d16 = dst_v[pl.ds(pos, L)]
            tb = et_v[pl.ds(pos, L)] * NPAD
            ga2[c, pl.ds(i * L, L)] = jnp.where(valid, s16, 0)
            key2[c, pl.ds(i * L, L)] = jnp.where(valid, tb + d16, GARB)
        return carry

    lax.fori_loop(0, NCH, idx_chunk, 0)
    dz1.wait()
    plsc.subcore_barrier()
    _stream_pass(out1_hbm, ga2, key2, rows, gsem, ssem, a_sh)
    plsc.subcore_barrier()
    pltpu.sync_copy(a_sh.at[pl.ds(rows0, BPT)],
                    a_out.at[cid, pl.ds(rows0, BPT)])


def _tcc_body(a_ref, inv_ref, out1_ref, w_ref, b2_ref, o_ref):
    p = inv_ref[...] * (a_ref[0] + a_ref[1])
    xcat = jnp.concatenate(
        [p[r] for r in range(R)] + [out1_ref[...]], axis=1)
    acc = jnp.dot(xcat, w_ref[...],
                  preferred_element_type=jnp.float32) + b2_ref[...]
    lanemod = lax.broadcasted_iota(jnp.int32, (PBLK, 128), 1) % H
    nxt = pltpu.roll(acc, 127, 1)
    m = jnp.maximum(acc, nxt)
    s = jnp.exp(acc - m) + jnp.exp(nxt - m)
    lse0 = m + jnp.log(s)
    lse = jnp.where(lanemod == 0, lse0, pltpu.roll(lse0, 1, 1))
    res = acc - lse
    parts = [res[:, H * j:H * j + C] for j in range(8)]
    o_ref[...] = jnp.stack(parts, axis=1).reshape(8 * PBLK, C)


def _tcc(a2, inv, out1, wstack, b2row):
    return pl.pallas_call(
        _tcc_body,
        grid=(NG,),
        in_specs=[
            pl.BlockSpec((NC, R, PBLK, 128), lambda i: (0, 0, i, 0)),
            pl.BlockSpec((R, PBLK, 128), lambda i: (0, i, 0)),
            pl.BlockSpec((PBLK, 128), lambda i: (i, 0)),
            pl.BlockSpec((128 * (R + 1), 128), lambda i: (0, 0)),
            pl.BlockSpec((1, 128), lambda i: (0, 0)),
        ],
        out_specs=pl.BlockSpec((8 * PBLK, C), lambda i: (i, 0)),
        out_shape=jax.ShapeDtypeStruct((N, C), jnp.float32),
    )(a2, inv, out1, wstack, b2row)


def kernel(x, edge_index, edge_type, W1, root1, b1, W2, root2, b2):
    f32 = jnp.float32
    x = x.astype(f32)
    ei = edge_index.astype(jnp.int32)
    et = edge_type.astype(jnp.int32)

    zeros_t = jnp.zeros((BPT, H), f32)
    ones_t = jnp.ones((CH, H), f32)

    wcat = jnp.concatenate(
        [W1.astype(f32)[r] for r in range(R)] + [root1.astype(f32)], axis=1)

    h1p, xr1p = _tca(x, wcat, b1.astype(f32).reshape(1, H))

    a1, cc = _sc1(h1p.reshape(NR, H), ei, et, zeros_t, ones_t)

    out1p, invp = _tcb(a1.reshape(NC, R, PROW, 128),
                       cc.reshape(NC, R, PROW, 128), xr1p)

    a2 = _sc2(out1p.reshape(NPAD, H), ei, et, zeros_t)

    eye8 = jnp.eye(8, dtype=f32)
    blocks = [jnp.kron(eye8, jnp.pad(W2.astype(f32)[r], ((0, 0), (0, H - C))))
              for r in range(R)]
    blocks.append(jnp.kron(eye8, jnp.pad(root2.astype(f32),
                                         ((0, 0), (0, H - C)))))
    wstack = jnp.concatenate(blocks, axis=0)
    b2row = jnp.tile(jnp.pad(b2.astype(f32), (0, H - C)), 8).reshape(1, 128)

    return _tcc(a2.reshape(NC, R, PROW, 128), invp, out1p, wstack, b2row)
